# baseline (speedup 1.0000x reference)
import jax
import jax.numpy as jnp
from jax import lax
from jax.experimental import pallas as pl
from jax.experimental.pallas import tpu as pltpu
from jax.experimental.pallas import tpu_sc as plsc

VOCAB = 100000
EMBED = 64
BATCH = 1024
CTX = 20

NC = 2
NS = 16
NW = NC * NS
IDX_PER_W = BATCH * CTX // NW
GCHUNK = 128
N_CHUNKS = IDX_PER_W // GCHUNK
PAIR = 2 * EMBED


def _sc_gather_body(emb2_hbm, x_hbm, rows_hbm, idx_v, rows_v, sem):
    wid = lax.axis_index("s") * NC + lax.axis_index("c")
    base = wid * IDX_PER_W
    pltpu.sync_copy(x_hbm.at[pl.ds(base, IDX_PER_W)], idx_v)
    copies = []
    for k in range(N_CHUNKS):
        copies.append(pltpu.async_copy(
            emb2_hbm.at[idx_v.at[pl.ds(k * GCHUNK, GCHUNK)]],
            rows_v.at[pl.ds(k * GCHUNK, GCHUNK)],
            sem))
    for cp in copies:
        cp.wait()
    pltpu.sync_copy(rows_v, rows_hbm.at[pl.ds(base, IDX_PER_W)])


def _sc_gather(emb2, half_idx):
    mesh = plsc.VectorSubcoreMesh(core_axis_name="c", subcore_axis_name="s")
    fn = pl.kernel(
        _sc_gather_body,
        mesh=mesh,
        out_type=jax.ShapeDtypeStruct((BATCH * CTX, PAIR), jnp.float32),
        scratch_types=[
            pltpu.VMEM((IDX_PER_W,), jnp.int32),
            pltpu.VMEM((IDX_PER_W, PAIR), jnp.float32),
            pltpu.SemaphoreType.DMA,
        ],
    )
    return fn(emb2, half_idx)


TV = 512


def _tc_body(rows_ref, x_ref, w_ref, b_ref, out_ref, h_ref):
    @pl.when(pl.program_id(0) == 0)
    def _():
        rows2 = rows_ref[...]
        par = (x_ref[...] & 1).astype(jnp.float32)
        lo = rows2[:, :, :EMBED]
        hi = rows2[:, :, EMBED:]
        rows = lo + par[:, :, None] * (hi - lo)
        ssq = jnp.sum(rows * rows, axis=-1, keepdims=True)
        norms = jnp.sqrt(ssq)
        scale = jnp.minimum(1.0, 1.0 / jnp.maximum(norms, 1e-7))
        scaled = rows * scale
        acc = scaled[:, 0, :]
        for t in range(1, CTX):
            acc = acc + scaled[:, t, :]
        h_ref[...] = acc * jnp.float32(1.0 / CTX)

    out_ref[...] = lax.dot_general(
        h_ref[...], w_ref[...],
        dimension_numbers=(((1,), (1,)), ((), ())),
        preferred_element_type=jnp.float32) + b_ref[...]


def _project(rows, x, W, b2d):
    return pl.pallas_call(
        _tc_body,
        grid=(pl.cdiv(VOCAB, TV),),
        in_specs=[
            pl.BlockSpec((BATCH, CTX, PAIR), lambda j: (0, 0, 0)),
            pl.BlockSpec((BATCH, CTX), lambda j: (0, 0)),
            pl.BlockSpec((TV, EMBED), lambda j: (j, 0)),
            pl.BlockSpec((1, TV), lambda j: (0, j)),
        ],
        out_specs=pl.BlockSpec((BATCH, TV), lambda j: (0, j)),
        out_shape=jax.ShapeDtypeStruct((BATCH, VOCAB), jnp.float32),
        scratch_shapes=[pltpu.VMEM((BATCH, EMBED), jnp.float32)],
    )(rows, x, W, b2d)


def kernel(x, emb, W, b):
    emb2 = emb.reshape(VOCAB // 2, PAIR)
    half_idx = (x >> 1).reshape(-1)
    rows = _sc_gather(emb2, half_idx)
    return _project(rows.reshape(BATCH, CTX, PAIR), x, W, b.reshape(1, VOCAB))


# speedup vs baseline: 1.1139x; 1.1139x over previous
import jax
import jax.numpy as jnp
from jax import lax
from jax.experimental import pallas as pl
from jax.experimental.pallas import tpu as pltpu
from jax.experimental.pallas import tpu_sc as plsc

VOCAB = 100000
EMBED = 64
BATCH = 1024
CTX = 20

NC = 2
NS = 16
NW = NC * NS
IDX_PER_W = BATCH * CTX // NW
GCHUNK = 128
N_CHUNKS = IDX_PER_W // GCHUNK
PAIR = 2 * EMBED


def _sc_gather_body(emb_hbm, x_hbm, rows_hbm, idx_v, rows_v, sem):
    wid = lax.axis_index("s") * NC + lax.axis_index("c")
    base = wid * IDX_PER_W
    pltpu.sync_copy(x_hbm.at[pl.ds(base, IDX_PER_W)], idx_v)
    copies = []
    for k in range(N_CHUNKS):
        copies.append(pltpu.async_copy(
            emb_hbm.at[idx_v.at[pl.ds(k * GCHUNK, GCHUNK)]],
            rows_v.at[pl.ds(k * GCHUNK, GCHUNK)],
            sem))
    for cp in copies:
        cp.wait()
    pltpu.sync_copy(rows_v, rows_hbm.at[pl.ds(base, IDX_PER_W)])


def _sc_gather(emb, x_flat):
    mesh = plsc.VectorSubcoreMesh(core_axis_name="c", subcore_axis_name="s")
    fn = pl.kernel(
        _sc_gather_body,
        mesh=mesh,
        out_type=jax.ShapeDtypeStruct((BATCH * CTX, EMBED), jnp.float32),
        scratch_types=[
            pltpu.VMEM((IDX_PER_W,), jnp.int32),
            pltpu.VMEM((IDX_PER_W, EMBED), jnp.float32),
            pltpu.SemaphoreType.DMA,
        ],
        compiler_params=pltpu.CompilerParams(use_tc_tiling_on_sc=False),
    )
    return fn(emb, x_flat)


TV = 1024


def _tc_body(rows_ref, w_ref, b_ref, out_ref, h_ref):
    @pl.when(pl.program_id(0) == 0)
    def _():
        rows = rows_ref[...]
        ssq = jnp.sum(rows * rows, axis=-1, keepdims=True)
        norms = jnp.sqrt(ssq)
        scale = jnp.minimum(1.0, 1.0 / jnp.maximum(norms, 1e-7))
        scaled = rows * scale
        acc = scaled[:, 0, :]
        for t in range(1, CTX):
            acc = acc + scaled[:, t, :]
        h_ref[...] = (acc * jnp.float32(1.0 / CTX)).astype(jnp.bfloat16)

    out_ref[...] = lax.dot_general(
        h_ref[...], w_ref[...].astype(jnp.bfloat16),
        dimension_numbers=(((1,), (1,)), ((), ())),
        preferred_element_type=jnp.float32) + b_ref[...]


def _project(rows, W, b2d):
    return pl.pallas_call(
        _tc_body,
        grid=(pl.cdiv(VOCAB, TV),),
        in_specs=[
            pl.BlockSpec((BATCH, CTX, EMBED), lambda j: (0, 0, 0)),
            pl.BlockSpec((TV, EMBED), lambda j: (j, 0)),
            pl.BlockSpec((1, TV), lambda j: (0, j)),
        ],
        out_specs=pl.BlockSpec((BATCH, TV), lambda j: (0, j)),
        out_shape=jax.ShapeDtypeStruct((BATCH, VOCAB), jnp.float32),
        scratch_shapes=[pltpu.VMEM((BATCH, EMBED), jnp.bfloat16)],
    )(rows, W, b2d)


def kernel(x, emb, W, b):
    rows = _sc_gather(emb, x.reshape(-1))
    return _project(rows.reshape(BATCH, CTX, EMBED), W, b.reshape(1, VOCAB))
